# baseline (speedup 1.0000x reference)
import functools

import jax
import jax.numpy as jnp
from jax import lax
from jax.experimental import pallas as pl
from jax.experimental.pallas import tpu as pltpu
from jax.experimental.pallas import tpu_sc as plsc


def _make_sc_add(B, S, D, NC, NS, L):
    NW = NC * NS
    rows_per_tile = S // NW
    chunk_rows = 32
    n_chunks = rows_per_tile // chunk_rows
    CW = chunk_rows * D

    mesh = plsc.VectorSubcoreMesh(core_axis_name="c", subcore_axis_name="s")

    @functools.partial(
        pl.kernel,
        mesh=mesh,
        out_type=jax.ShapeDtypeStruct((B * S * D,), jnp.float32),
        scratch_types=[
            pltpu.VMEM((CW,), jnp.float32),
            pltpu.VMEM((CW,), jnp.float32),
        ],
    )
    def sc_add(x_hbm, emb_hbm, out_hbm, xbuf, ebuf):
        wid = lax.axis_index("s") * NC + lax.axis_index("c")
        base = wid * rows_per_tile * D
        for c in range(n_chunks):
            e_off = base + c * CW
            pltpu.sync_copy(emb_hbm.at[pl.ds(e_off, CW)], ebuf)
            for b in range(B):
                x_off = b * S * D + e_off
                pltpu.sync_copy(x_hbm.at[pl.ds(x_off, CW)], xbuf)

                @plsc.parallel_loop(0, CW, L, unroll=8)
                def _(i):
                    xbuf[pl.ds(i, L)] = xbuf[pl.ds(i, L)] + ebuf[pl.ds(i, L)]

                pltpu.sync_copy(xbuf, out_hbm.at[pl.ds(x_off, CW)])

    return sc_add


def kernel(x, emb):
    B, S, D = x.shape
    info = plsc.get_sparse_core_info()
    NC, NS, L = info.num_cores, info.num_subcores, info.num_lanes
    sc_add = _make_sc_add(B, S, D, NC, NS, L)
    out = sc_add(x.reshape(B * S * D), emb[:S].reshape(S * D))
    return out.reshape(B, S, D)


# speedup vs baseline: 1.2124x; 1.2124x over previous
import functools

import jax
import jax.numpy as jnp
from jax import lax
from jax.experimental import pallas as pl
from jax.experimental.pallas import tpu as pltpu
from jax.experimental.pallas import tpu_sc as plsc


def _make_sc_add(B, S, D, NC, NS, L):
    NW = NC * NS
    rows_per_tile = S // NW
    chunk_rows = 32
    n_chunks = rows_per_tile // chunk_rows
    CW = chunk_rows * D
    NSLOT = 3
    steps = [(c, b) for c in range(n_chunks) for b in range(B)]
    n_steps = len(steps)

    mesh = plsc.VectorSubcoreMesh(core_axis_name="c", subcore_axis_name="s")

    @functools.partial(
        pl.kernel,
        mesh=mesh,
        out_type=jax.ShapeDtypeStruct((B * S * D,), jnp.float32),
        scratch_types=[
            pltpu.VMEM((CW,), jnp.float32),
            pltpu.VMEM((CW,), jnp.float32),
            pltpu.VMEM((CW,), jnp.float32),
            pltpu.VMEM((CW,), jnp.float32),
            pltpu.VMEM((CW,), jnp.float32),
            pltpu.SemaphoreType.DMA,
            pltpu.SemaphoreType.DMA,
            pltpu.SemaphoreType.DMA,
            pltpu.SemaphoreType.DMA,
            pltpu.SemaphoreType.DMA,
            pltpu.SemaphoreType.DMA,
            pltpu.SemaphoreType.DMA,
            pltpu.SemaphoreType.DMA,
        ],
    )
    def sc_add(
        x_hbm, emb_hbm, out_hbm, xb0, xb1, xb2, eb0, eb1,
        i0, i1, i2, o0, o1, o2, e0, e1,
    ):
        xbuf = [xb0, xb1, xb2]
        ebuf = [eb0, eb1]
        insem = [i0, i1, i2]
        outsem = [o0, o1, o2]
        esem = [e0, e1]
        wid = lax.axis_index("s") * NC + lax.axis_index("c")
        base = wid * rows_per_tile * D

        def x_off(c, b):
            return b * S * D + c * CW

        def in_copy(i):
            c, b = steps[i]
            slot = i % NSLOT
            return pltpu.make_async_copy(
                x_hbm.at[pl.ds(base + x_off(c, b), CW)], xbuf[slot], insem[slot]
            )

        def out_copy(i):
            c, b = steps[i]
            slot = i % NSLOT
            return pltpu.make_async_copy(
                xbuf[slot], out_hbm.at[pl.ds(base + x_off(c, b), CW)], outsem[slot]
            )

        def e_copy(c):
            return pltpu.make_async_copy(
                emb_hbm.at[pl.ds(base + c * CW, CW)], ebuf[c % 2], esem[c % 2]
            )

        e_copy(0).start()
        in_copy(0).start()
        in_copy(1).start()
        for i in range(n_steps):
            c, b = steps[i]
            slot = i % NSLOT
            j = i + 2
            if j < n_steps:
                if j >= NSLOT:
                    out_copy(j - NSLOT).wait()
                in_copy(j).start()
            if b == 0 and c + 1 < n_chunks:
                e_copy(c + 1).start()
            in_copy(i).wait()
            if b == 0:
                e_copy(c).wait()
            xb = xbuf[slot]
            eb = ebuf[c % 2]

            @plsc.parallel_loop(0, CW, L, unroll=8)
            def _(k):
                xb[pl.ds(k, L)] = xb[pl.ds(k, L)] + eb[pl.ds(k, L)]

            out_copy(i).start()
        for i in range(max(0, n_steps - NSLOT), n_steps):
            out_copy(i).wait()

    return sc_add


def kernel(x, emb):
    B, S, D = x.shape
    info = plsc.get_sparse_core_info()
    NC, NS, L = info.num_cores, info.num_subcores, info.num_lanes
    sc_add = _make_sc_add(B, S, D, NC, NS, L)
    out = sc_add(x.reshape(B * S * D), emb[:S].reshape(S * D))
    return out.reshape(B, S, D)


# speedup vs baseline: 3.3043x; 2.7254x over previous
import functools

import jax
import jax.numpy as jnp
from jax import lax
from jax.experimental import pallas as pl
from jax.experimental.pallas import tpu as pltpu
from jax.experimental.pallas import tpu_sc as plsc


def _make_sc_add(B, S, D, NC, NS, L):
    NW = NC * NS
    rows_per_tile = S // NW
    chunk_rows = 32
    n_chunks = rows_per_tile // chunk_rows
    NSLOT = 3
    steps = [(c, b) for c in range(n_chunks) for b in range(B)]
    n_steps = len(steps)

    mesh = plsc.VectorSubcoreMesh(core_axis_name="c", subcore_axis_name="s")

    @functools.partial(
        pl.kernel,
        mesh=mesh,
        out_type=jax.ShapeDtypeStruct((B * S, D), jnp.float32),
        scratch_types=[
            pltpu.VMEM((chunk_rows, D), jnp.float32),
            pltpu.VMEM((chunk_rows, D), jnp.float32),
            pltpu.VMEM((chunk_rows, D), jnp.float32),
            pltpu.VMEM((chunk_rows, D), jnp.float32),
            pltpu.VMEM((chunk_rows, D), jnp.float32),
            pltpu.SemaphoreType.DMA,
            pltpu.SemaphoreType.DMA,
            pltpu.SemaphoreType.DMA,
            pltpu.SemaphoreType.DMA,
            pltpu.SemaphoreType.DMA,
            pltpu.SemaphoreType.DMA,
            pltpu.SemaphoreType.DMA,
            pltpu.SemaphoreType.DMA,
        ],
    )
    def sc_add(
        x_hbm, emb_hbm, out_hbm, xb0, xb1, xb2, eb0, eb1,
        i0, i1, i2, o0, o1, o2, e0, e1,
    ):
        xbuf = [xb0, xb1, xb2]
        ebuf = [eb0, eb1]
        insem = [i0, i1, i2]
        outsem = [o0, o1, o2]
        esem = [e0, e1]
        wid = lax.axis_index("s") * NC + lax.axis_index("c")
        base = wid * rows_per_tile

        def row0(c, b):
            return b * S + base + c * chunk_rows

        def in_copy(i):
            c, b = steps[i]
            slot = i % NSLOT
            return pltpu.make_async_copy(
                x_hbm.at[pl.ds(row0(c, b), chunk_rows)], xbuf[slot], insem[slot]
            )

        def out_copy(i):
            c, b = steps[i]
            slot = i % NSLOT
            return pltpu.make_async_copy(
                xbuf[slot], out_hbm.at[pl.ds(row0(c, b), chunk_rows)], outsem[slot]
            )

        def e_copy(c):
            return pltpu.make_async_copy(
                emb_hbm.at[pl.ds(base + c * chunk_rows, chunk_rows)],
                ebuf[c % 2],
                esem[c % 2],
            )

        e_copy(0).start()
        in_copy(0).start()
        in_copy(1).start()
        for i in range(n_steps):
            c, b = steps[i]
            slot = i % NSLOT
            j = i + 2
            if j < n_steps:
                if j >= NSLOT:
                    out_copy(j - NSLOT).wait()
                in_copy(j).start()
            if b == 0 and c + 1 < n_chunks:
                e_copy(c + 1).start()
            in_copy(i).wait()
            if b == 0:
                e_copy(c).wait()
            xb = xbuf[slot]
            eb = ebuf[c % 2]

            @plsc.parallel_loop(0, chunk_rows, 1)
            def _(r):
                @plsc.parallel_loop(0, D, L, unroll=8)
                def _(k):
                    xb[r, pl.ds(k, L)] = xb[r, pl.ds(k, L)] + eb[r, pl.ds(k, L)]

            out_copy(i).start()
        for i in range(max(0, n_steps - NSLOT), n_steps):
            out_copy(i).wait()

    return sc_add


def kernel(x, emb):
    B, S, D = x.shape
    info = plsc.get_sparse_core_info()
    NC, NS, L = info.num_cores, info.num_subcores, info.num_lanes
    sc_add = _make_sc_add(B, S, D, NC, NS, L)
    out = sc_add(x.reshape(B * S, D), emb[:S])
    return out.reshape(B, S, D)
